# baseline (speedup 1.0000x reference)
import functools

import jax
import jax.numpy as jnp
from jax import lax
from jax.experimental import pallas as pl
from jax.experimental.pallas import tpu as pltpu
from jax.experimental.pallas import tpu_sc as plsc

N_NODES = 10000
N_EDGES = 320000
D = 128

NUM_CORES = 2
NUM_SUBCORES = 16
NW = NUM_CORES * NUM_SUBCORES
CHUNK = 128
ROWS_C0 = 53
ROWS_C1 = 105
TOTAL_CHUNKS = NUM_SUBCORES * (ROWS_C0 + ROWS_C1)
E_PAD = TOTAL_CHUNKS * CHUNK
ACC_ROWS = 10240
ZERO_ROWS = ACC_ROWS // NUM_SUBCORES


def _sc_segment_sum(feature, idxp, zeros):
    mesh = plsc.VectorSubcoreMesh(core_axis_name="c", subcore_axis_name="s")

    @functools.partial(
        pl.kernel,
        out_type=jax.ShapeDtypeStruct((NUM_CORES, ACC_ROWS, D), jnp.float32),
        mesh=mesh,
        scratch_types=[
            pltpu.VMEM((2, CHUNK), jnp.int32),
            pltpu.VMEM((2, CHUNK), jnp.int32),
            pltpu.VMEM((CHUNK, D), jnp.float32),
            pltpu.VMEM((CHUNK, D), jnp.float32),
            pltpu.VMEM_SHARED((ACC_ROWS, D), jnp.float32),
            pltpu.SemaphoreType.DMA,
            pltpu.SemaphoreType.DMA,
            pltpu.SemaphoreType.DMA,
        ],
    )
    def k(feature_hbm, idx_hbm, zeros_hbm, out_hbm,
          ibuf0, ibuf1, buf0, buf1, acc, isem, gsem, ssem):
        c = lax.axis_index("c")
        s = lax.axis_index("s")
        n = jnp.where(c == 0, ROWS_C0, ROWS_C1)
        off = jnp.where(c == 0, s * ROWS_C0,
                        NUM_SUBCORES * ROWS_C0 + s * ROWS_C1)
        pltpu.sync_copy(zeros_hbm, acc.at[pl.ds(s * ZERO_ROWS, ZERO_ROWS)])
        plsc.subcore_barrier()

        def fetch_idx(j, ibuf):
            pltpu.async_copy(idx_hbm.at[off + j], ibuf, isem)

        def wait_idx(ibuf):
            pltpu.make_async_copy(idx_hbm.at[0], ibuf, isem).wait()

        def gather(ibuf, buf):
            pltpu.async_copy(feature_hbm.at[ibuf.at[0]], buf, gsem).wait()

        def scatter(ibuf, buf):
            pltpu.async_copy(buf, acc.at[ibuf.at[1]], ssem, add=True)

        def wait_scatter(ibuf, buf):
            pltpu.make_async_copy(buf, acc.at[ibuf.at[1]], ssem).wait()

        fetch_idx(0, ibuf0)

        def body(i, carry):
            j = 2 * i
            wait_idx(ibuf0)
            gather(ibuf0, buf0)
            scatter(ibuf0, buf0)

            @pl.when(i > 0)
            def _():
                wait_scatter(ibuf1, buf1)

            fetch_idx(j + 1, ibuf1)
            wait_idx(ibuf1)
            gather(ibuf1, buf1)
            scatter(ibuf1, buf1)
            wait_scatter(ibuf0, buf0)
            fetch_idx(jnp.minimum(j + 2, n - 1), ibuf0)
            return carry

        lax.fori_loop(0, (n - 1) // 2, body, 0)
        wait_idx(ibuf0)
        gather(ibuf0, buf0)
        scatter(ibuf0, buf0)
        wait_scatter(ibuf1, buf1)
        wait_scatter(ibuf0, buf0)
        plsc.subcore_barrier()
        pltpu.sync_copy(acc.at[pl.ds(s * ZERO_ROWS, ZERO_ROWS)],
                        out_hbm.at[c, pl.ds(s * ZERO_ROWS, ZERO_ROWS)])

    return k(feature, idxp, zeros)


def _tc_linear(partials, W, b):
    blk = 1000
    grid = N_NODES // blk

    def body(p_ref, w_ref, b_ref, o_ref):
        x = p_ref[0] + p_ref[1]
        y = lax.dot_general(x, w_ref[...], (((1,), (1,)), ((), ())),
                            preferred_element_type=jnp.float32,
                            precision=lax.Precision.HIGHEST)
        o_ref[...] = y + b_ref[...]

    return pl.pallas_call(
        body,
        grid=(grid,),
        in_specs=[
            pl.BlockSpec((NUM_CORES, blk, D), lambda i: (0, i, 0)),
            pl.BlockSpec((D, D), lambda i: (0, 0)),
            pl.BlockSpec((1, D), lambda i: (0, 0)),
        ],
        out_specs=pl.BlockSpec((blk, D), lambda i: (i, 0)),
        out_shape=jax.ShapeDtypeStruct((N_NODES, D), jnp.float32),
    )(partials, W, b.reshape(1, D))


def kernel(feature, edge_index, W, b):
    src = edge_index[0].astype(jnp.int32)
    dst = edge_index[1].astype(jnp.int32)
    pad = E_PAD - N_EDGES
    src = jnp.concatenate([src, jnp.zeros((pad,), jnp.int32)])
    dst = jnp.concatenate([dst, jnp.full((pad,), N_NODES, jnp.int32)])
    idxp = jnp.stack([src.reshape(TOTAL_CHUNKS, CHUNK),
                      dst.reshape(TOTAL_CHUNKS, CHUNK)], axis=1)
    zeros = jnp.zeros((ZERO_ROWS, D), jnp.float32)
    partials = _sc_segment_sum(feature, idxp, zeros)
    return _tc_linear(partials, W, b)


# speedup vs baseline: 1.2424x; 1.2424x over previous
import functools

import jax
import jax.numpy as jnp
from jax import lax
from jax.experimental import pallas as pl
from jax.experimental.pallas import tpu as pltpu
from jax.experimental.pallas import tpu_sc as plsc

N_NODES = 10000
N_EDGES = 320000
D = 128

NUM_CORES = 2
NUM_SUBCORES = 16
NW = NUM_CORES * NUM_SUBCORES
CHUNK = 128
ROWS_C0 = 105
ROWS_C1 = 53
TOTAL_CHUNKS = NUM_SUBCORES * (ROWS_C0 + ROWS_C1)
E_PAD = TOTAL_CHUNKS * CHUNK
ACC_ROWS = 10240
ZERO_ROWS = ACC_ROWS // NUM_SUBCORES


def _sc_segment_sum(feature, idxp, zeros):
    mesh = plsc.VectorSubcoreMesh(core_axis_name="c", subcore_axis_name="s")

    @functools.partial(
        pl.kernel,
        out_type=jax.ShapeDtypeStruct((NUM_CORES, ACC_ROWS, D), jnp.float32),
        mesh=mesh,
        scratch_types=[
            pltpu.VMEM((2, CHUNK), jnp.int32),
            pltpu.VMEM((2, CHUNK), jnp.int32),
            pltpu.VMEM((CHUNK, D), jnp.float32),
            pltpu.VMEM((CHUNK, D), jnp.float32),
            pltpu.VMEM_SHARED((ACC_ROWS, D), jnp.float32),
            pltpu.SemaphoreType.DMA,
            pltpu.SemaphoreType.DMA,
            pltpu.SemaphoreType.DMA,
        ],
    )
    def k(feature_hbm, idx_hbm, zeros_hbm, out_hbm,
          ibuf0, ibuf1, buf0, buf1, acc, isem, gsem, ssem):
        c = lax.axis_index("c")
        s = lax.axis_index("s")
        n = jnp.where(c == 0, ROWS_C0, ROWS_C1)
        off = jnp.where(c == 0, s * ROWS_C0,
                        NUM_SUBCORES * ROWS_C0 + s * ROWS_C1)
        pltpu.sync_copy(zeros_hbm, acc.at[pl.ds(s * ZERO_ROWS, ZERO_ROWS)])
        plsc.subcore_barrier()

        def fetch_idx(j, ibuf):
            pltpu.async_copy(idx_hbm.at[off + j], ibuf, isem)

        def wait_idx(ibuf):
            pltpu.make_async_copy(idx_hbm.at[0], ibuf, isem).wait()

        def gather(ibuf, buf):
            pltpu.async_copy(feature_hbm.at[ibuf.at[0]], buf, gsem).wait()

        def scatter(ibuf, buf):
            pltpu.async_copy(buf, acc.at[ibuf.at[1]], ssem, add=True)

        def wait_scatter(ibuf, buf):
            pltpu.make_async_copy(buf, acc.at[ibuf.at[1]], ssem).wait()

        fetch_idx(0, ibuf0)

        def body(i, carry):
            j = 2 * i
            wait_idx(ibuf0)
            gather(ibuf0, buf0)
            scatter(ibuf0, buf0)

            @pl.when(i > 0)
            def _():
                wait_scatter(ibuf1, buf1)

            fetch_idx(j + 1, ibuf1)
            wait_idx(ibuf1)
            gather(ibuf1, buf1)
            scatter(ibuf1, buf1)
            wait_scatter(ibuf0, buf0)
            fetch_idx(jnp.minimum(j + 2, n - 1), ibuf0)
            return carry

        lax.fori_loop(0, (n - 1) // 2, body, 0)
        wait_idx(ibuf0)
        gather(ibuf0, buf0)
        scatter(ibuf0, buf0)
        wait_scatter(ibuf1, buf1)
        wait_scatter(ibuf0, buf0)
        plsc.subcore_barrier()
        pltpu.sync_copy(acc.at[pl.ds(s * ZERO_ROWS, ZERO_ROWS)],
                        out_hbm.at[c, pl.ds(s * ZERO_ROWS, ZERO_ROWS)])

    return k(feature, idxp, zeros)


def _tc_linear(partials, W, b):
    blk = 1000
    grid = N_NODES // blk

    def body(p_ref, w_ref, b_ref, o_ref):
        x = p_ref[0] + p_ref[1]
        y = lax.dot_general(x, w_ref[...], (((1,), (1,)), ((), ())),
                            preferred_element_type=jnp.float32,
                            precision=lax.Precision.HIGHEST)
        o_ref[...] = y + b_ref[...]

    return pl.pallas_call(
        body,
        grid=(grid,),
        in_specs=[
            pl.BlockSpec((NUM_CORES, blk, D), lambda i: (0, i, 0)),
            pl.BlockSpec((D, D), lambda i: (0, 0)),
            pl.BlockSpec((1, D), lambda i: (0, 0)),
        ],
        out_specs=pl.BlockSpec((blk, D), lambda i: (i, 0)),
        out_shape=jax.ShapeDtypeStruct((N_NODES, D), jnp.float32),
    )(partials, W, b.reshape(1, D))


def kernel(feature, edge_index, W, b):
    src = edge_index[0].astype(jnp.int32)
    dst = edge_index[1].astype(jnp.int32)
    pad = E_PAD - N_EDGES
    src = jnp.concatenate([src, jnp.zeros((pad,), jnp.int32)])
    dst = jnp.concatenate([dst, jnp.full((pad,), N_NODES, jnp.int32)])
    idxp = jnp.stack([src.reshape(TOTAL_CHUNKS, CHUNK),
                      dst.reshape(TOTAL_CHUNKS, CHUNK)], axis=1)
    zeros = jnp.zeros((ZERO_ROWS, D), jnp.float32)
    partials = _sc_segment_sum(feature, idxp, zeros)
    return _tc_linear(partials, W, b)


# speedup vs baseline: 1.3103x; 1.0547x over previous
import functools

import jax
import jax.numpy as jnp
from jax import lax
from jax.experimental import pallas as pl
from jax.experimental.pallas import tpu as pltpu
from jax.experimental.pallas import tpu_sc as plsc

N_NODES = 10000
N_EDGES = 320000
D = 128

NUM_CORES = 2
NUM_SUBCORES = 16
NW = NUM_CORES * NUM_SUBCORES
CHUNK = 128
ROWS_C0 = 115
ROWS_C1 = 43
TOTAL_CHUNKS = NUM_SUBCORES * (ROWS_C0 + ROWS_C1)
E_PAD = TOTAL_CHUNKS * CHUNK
ACC_ROWS = 10240
ZERO_ROWS = ACC_ROWS // NUM_SUBCORES


def _sc_segment_sum(feature, srcp, dstp, zeros):
    mesh = plsc.VectorSubcoreMesh(core_axis_name="c", subcore_axis_name="s")

    @functools.partial(
        pl.kernel,
        out_type=jax.ShapeDtypeStruct((NUM_CORES, ACC_ROWS, D), jnp.float32),
        mesh=mesh,
        scratch_types=[
            pltpu.VMEM((2, CHUNK), jnp.int32),
            pltpu.VMEM((2, CHUNK), jnp.int32),
            pltpu.VMEM((CHUNK, D), jnp.float32),
            pltpu.VMEM((CHUNK, D), jnp.float32),
            pltpu.VMEM_SHARED((ACC_ROWS, D), jnp.float32),
            pltpu.SemaphoreType.DMA,
            pltpu.SemaphoreType.DMA,
            pltpu.SemaphoreType.DMA,
        ],
    )
    def k(feature_hbm, src_hbm, dst_hbm, zeros_hbm, out_hbm,
          ibuf0, ibuf1, buf0, buf1, acc, isem, gsem, ssem):
        c = lax.axis_index("c")
        s = lax.axis_index("s")
        n = jnp.where(c == 0, ROWS_C0, ROWS_C1)
        off = jnp.where(c == 0, s * ROWS_C0,
                        NUM_SUBCORES * ROWS_C0 + s * ROWS_C1)
        pltpu.sync_copy(zeros_hbm, acc.at[pl.ds(s * ZERO_ROWS, ZERO_ROWS)])
        plsc.subcore_barrier()

        def fetch_idx(j, ibuf):
            e = (off + j) * CHUNK
            pltpu.async_copy(src_hbm.at[pl.ds(e, CHUNK)], ibuf.at[0], isem)
            pltpu.async_copy(dst_hbm.at[pl.ds(e, CHUNK)], ibuf.at[1], isem)

        def wait_idx(ibuf):
            pltpu.make_async_copy(src_hbm.at[pl.ds(0, CHUNK)],
                                  ibuf.at[0], isem).wait()
            pltpu.make_async_copy(dst_hbm.at[pl.ds(0, CHUNK)],
                                  ibuf.at[1], isem).wait()

        def gather(ibuf, buf):
            pltpu.async_copy(feature_hbm.at[ibuf.at[0]], buf, gsem).wait()

        def scatter(ibuf, buf):
            pltpu.async_copy(buf, acc.at[ibuf.at[1]], ssem, add=True)

        def wait_scatter(ibuf, buf):
            pltpu.make_async_copy(buf, acc.at[ibuf.at[1]], ssem).wait()

        fetch_idx(0, ibuf0)

        def body(i, carry):
            j = 2 * i
            wait_idx(ibuf0)
            gather(ibuf0, buf0)
            scatter(ibuf0, buf0)

            @pl.when(i > 0)
            def _():
                wait_scatter(ibuf1, buf1)

            fetch_idx(j + 1, ibuf1)
            wait_idx(ibuf1)
            gather(ibuf1, buf1)
            scatter(ibuf1, buf1)
            wait_scatter(ibuf0, buf0)
            fetch_idx(jnp.minimum(j + 2, n - 1), ibuf0)
            return carry

        lax.fori_loop(0, (n - 1) // 2, body, 0)
        wait_idx(ibuf0)
        gather(ibuf0, buf0)
        scatter(ibuf0, buf0)
        wait_scatter(ibuf1, buf1)
        wait_scatter(ibuf0, buf0)
        plsc.subcore_barrier()
        pltpu.sync_copy(acc.at[pl.ds(s * ZERO_ROWS, ZERO_ROWS)],
                        out_hbm.at[c, pl.ds(s * ZERO_ROWS, ZERO_ROWS)])

    return k(feature, srcp, dstp, zeros)


def _tc_linear(partials, W, b):
    blk = 1000
    grid = N_NODES // blk

    def body(p_ref, w_ref, b_ref, o_ref):
        x = p_ref[0] + p_ref[1]
        y = lax.dot_general(x, w_ref[...], (((1,), (1,)), ((), ())),
                            preferred_element_type=jnp.float32,
                            precision=lax.Precision.HIGHEST)
        o_ref[...] = y + b_ref[...]

    return pl.pallas_call(
        body,
        grid=(grid,),
        in_specs=[
            pl.BlockSpec((NUM_CORES, blk, D), lambda i: (0, i, 0)),
            pl.BlockSpec((D, D), lambda i: (0, 0)),
            pl.BlockSpec((1, D), lambda i: (0, 0)),
        ],
        out_specs=pl.BlockSpec((blk, D), lambda i: (i, 0)),
        out_shape=jax.ShapeDtypeStruct((N_NODES, D), jnp.float32),
    )(partials, W, b.reshape(1, D))


def kernel(feature, edge_index, W, b):
    src = edge_index[0].astype(jnp.int32)
    dst = edge_index[1].astype(jnp.int32)
    pad = E_PAD - N_EDGES
    srcp = jnp.pad(src, (0, pad))
    dstp = jnp.pad(dst, (0, pad), constant_values=N_NODES)
    zeros = jnp.zeros((ZERO_ROWS, D), jnp.float32)
    partials = _sc_segment_sum(feature, srcp, dstp, zeros)
    return _tc_linear(partials, W, b)


# speedup vs baseline: 1.6095x; 1.2283x over previous
import functools

import jax
import jax.numpy as jnp
from jax import lax
from jax.experimental import pallas as pl
from jax.experimental.pallas import tpu as pltpu
from jax.experimental.pallas import tpu_sc as plsc

N_NODES = 10000
N_EDGES = 320000
D = 128

NUM_CORES = 2
NUM_SUBCORES = 16
NW = NUM_CORES * NUM_SUBCORES
CHUNK = 128
ROWS_C0 = 115
ROWS_C1 = 43
TOTAL_CHUNKS = NUM_SUBCORES * (ROWS_C0 + ROWS_C1)
E_PAD = TOTAL_CHUNKS * CHUNK
ACC_ROWS = 10240
ZERO_ROWS = ACC_ROWS // NUM_SUBCORES


def _sc_segment_sum(feature, srcp, dstp, zeros):
    mesh = plsc.VectorSubcoreMesh(core_axis_name="c", subcore_axis_name="s")

    @functools.partial(
        pl.kernel,
        out_type=jax.ShapeDtypeStruct((NUM_CORES, ACC_ROWS, D), jnp.float32),
        mesh=mesh,
        scratch_types=[
            pltpu.VMEM((2, CHUNK), jnp.int32),
            pltpu.VMEM((2, CHUNK), jnp.int32),
            pltpu.VMEM((CHUNK, D), jnp.float32),
            pltpu.VMEM((CHUNK, D), jnp.float32),
            pltpu.VMEM_SHARED((ACC_ROWS, D), jnp.float32),
            pltpu.SemaphoreType.DMA,
            pltpu.SemaphoreType.DMA,
            pltpu.SemaphoreType.DMA,
        ],
    )
    def k(feature_hbm, src_hbm, dst_hbm, zeros_hbm, out_hbm,
          ibuf0, ibuf1, buf0, buf1, acc, isem, gsem, ssem):
        c = lax.axis_index("c")
        s = lax.axis_index("s")
        n = jnp.where(c == 0, ROWS_C0, ROWS_C1)
        off = jnp.where(c == 0, s * ROWS_C0,
                        NUM_SUBCORES * ROWS_C0 + s * ROWS_C1)
        pltpu.sync_copy(zeros_hbm, acc.at[pl.ds(s * ZERO_ROWS, ZERO_ROWS)])
        plsc.subcore_barrier()

        def fetch_idx(j, ibuf):
            e = (off + j) * CHUNK
            pltpu.async_copy(src_hbm.at[pl.ds(e, CHUNK)], ibuf.at[0], isem)
            pltpu.async_copy(dst_hbm.at[pl.ds(e, CHUNK)], ibuf.at[1], isem)

        def wait_idx(ibuf):
            pltpu.make_async_copy(src_hbm.at[pl.ds(0, CHUNK)],
                                  ibuf.at[0], isem).wait()
            pltpu.make_async_copy(dst_hbm.at[pl.ds(0, CHUNK)],
                                  ibuf.at[1], isem).wait()

        def gather(ibuf, buf):
            pltpu.async_copy(feature_hbm.at[ibuf.at[0]], buf, gsem).wait()

        def scatter(ibuf, buf):
            pltpu.async_copy(buf, acc.at[ibuf.at[1]], ssem, add=True)

        def wait_scatter(ibuf, buf):
            pltpu.make_async_copy(buf, acc.at[ibuf.at[1]], ssem).wait()

        fetch_idx(0, ibuf0)

        def body(i, carry):
            j = 2 * i
            wait_idx(ibuf0)
            gather(ibuf0, buf0)
            scatter(ibuf0, buf0)

            @pl.when(i > 0)
            def _():
                wait_scatter(ibuf1, buf1)

            fetch_idx(j + 1, ibuf1)
            wait_idx(ibuf1)
            gather(ibuf1, buf1)
            scatter(ibuf1, buf1)
            wait_scatter(ibuf0, buf0)
            fetch_idx(jnp.minimum(j + 2, n - 1), ibuf0)
            return carry

        lax.fori_loop(0, (n - 1) // 2, body, 0)
        wait_idx(ibuf0)
        gather(ibuf0, buf0)
        scatter(ibuf0, buf0)
        wait_scatter(ibuf1, buf1)
        wait_scatter(ibuf0, buf0)
        plsc.subcore_barrier()
        pltpu.sync_copy(acc.at[pl.ds(s * ZERO_ROWS, ZERO_ROWS)],
                        out_hbm.at[c, pl.ds(s * ZERO_ROWS, ZERO_ROWS)])

    return k(feature, srcp, dstp, zeros)


def _tc_linear(partials, W, b):
    blk = 1000
    grid = N_NODES // blk

    def body(p_ref, w_ref, b_ref, o_ref):
        x = p_ref[0] + p_ref[1]
        y = lax.dot_general(x, w_ref[...], (((1,), (1,)), ((), ())),
                            preferred_element_type=jnp.float32,
                            precision=lax.Precision.HIGHEST)
        o_ref[...] = y + b_ref[...]

    return pl.pallas_call(
        body,
        grid=(grid,),
        in_specs=[
            pl.BlockSpec((NUM_CORES, blk, D), lambda i: (0, i, 0)),
            pl.BlockSpec((D, D), lambda i: (0, 0)),
            pl.BlockSpec((1, D), lambda i: (0, 0)),
        ],
        out_specs=pl.BlockSpec((blk, D), lambda i: (i, 0)),
        out_shape=jax.ShapeDtypeStruct((N_NODES, D), jnp.float32),
    )(partials, W, b.reshape(1, D))


def kernel(feature, edge_index, W, b):
    src = edge_index[0].astype(jnp.int32)
    dst = edge_index[1].astype(jnp.int32)
    pad = E_PAD - N_EDGES
    pad_src = jnp.arange(pad, dtype=jnp.int32) % N_NODES
    pad_dst = (N_NODES
               + jnp.arange(pad, dtype=jnp.int32) % (ACC_ROWS - N_NODES))
    srcp = jnp.concatenate([src, pad_src])
    dstp = jnp.concatenate([dst, pad_dst])
    zeros = jnp.zeros((ZERO_ROWS, D), jnp.float32)
    partials = _sc_segment_sum(feature, srcp, dstp, zeros)
    return _tc_linear(partials, W, b)


# speedup vs baseline: 1.6915x; 1.0509x over previous
import functools

import jax
import jax.numpy as jnp
from jax import lax
from jax.experimental import pallas as pl
from jax.experimental.pallas import tpu as pltpu
from jax.experimental.pallas import tpu_sc as plsc

N_NODES = 10000
N_EDGES = 320000
D = 128

NUM_CORES = 2
NUM_SUBCORES = 16
NW = NUM_CORES * NUM_SUBCORES
CHUNK = 128
ROWS_C0 = 115
ROWS_C1 = 43
TOTAL_CHUNKS = NUM_SUBCORES * (ROWS_C0 + ROWS_C1)
E_PAD = TOTAL_CHUNKS * CHUNK
ACC_ROWS = 10240
ZERO_ROWS = ACC_ROWS // NUM_SUBCORES


def _sc_segment_sum(feature, ei, pads, zeros):
    mesh = plsc.VectorSubcoreMesh(core_axis_name="c", subcore_axis_name="s")

    @functools.partial(
        pl.kernel,
        out_type=jax.ShapeDtypeStruct((NUM_CORES, ACC_ROWS, D), jnp.float32),
        mesh=mesh,
        scratch_types=[
            pltpu.VMEM((2, CHUNK), jnp.int32),
            pltpu.VMEM((2, CHUNK), jnp.int32),
            pltpu.VMEM((CHUNK, D), jnp.float32),
            pltpu.VMEM((CHUNK, D), jnp.float32),
            pltpu.VMEM_SHARED((ACC_ROWS, D), jnp.float32),
            pltpu.SemaphoreType.DMA,
            pltpu.SemaphoreType.DMA,
            pltpu.SemaphoreType.DMA,
        ],
    )
    def k(feature_hbm, ei_hbm, pads_hbm, zeros_hbm, out_hbm,
          ibuf0, ibuf1, buf0, buf1, acc, isem, gsem, ssem):
        c = lax.axis_index("c")
        s = lax.axis_index("s")
        n = jnp.where(c == 0, ROWS_C0, ROWS_C1)
        off = jnp.where(c == 0, s * ROWS_C0,
                        NUM_SUBCORES * ROWS_C0 + s * ROWS_C1)
        pltpu.sync_copy(zeros_hbm, acc.at[pl.ds(s * ZERO_ROWS, ZERO_ROWS)])
        plsc.subcore_barrier()

        def fetch_idx(j, ibuf):
            e = (off + j) * CHUNK

            @pl.when(e < N_EDGES)
            def _():
                pltpu.async_copy(ei_hbm.at[0, pl.ds(e, CHUNK)],
                                 ibuf.at[0], isem)
                pltpu.async_copy(ei_hbm.at[1, pl.ds(e, CHUNK)],
                                 ibuf.at[1], isem)

            @pl.when(e >= N_EDGES)
            def _():
                ep = e - N_EDGES
                pltpu.async_copy(pads_hbm.at[0, pl.ds(ep, CHUNK)],
                                 ibuf.at[0], isem)
                pltpu.async_copy(pads_hbm.at[1, pl.ds(ep, CHUNK)],
                                 ibuf.at[1], isem)

        def wait_idx(ibuf):
            pltpu.make_async_copy(ei_hbm.at[0, pl.ds(0, CHUNK)],
                                  ibuf.at[0], isem).wait()
            pltpu.make_async_copy(ei_hbm.at[1, pl.ds(0, CHUNK)],
                                  ibuf.at[1], isem).wait()

        def gather(ibuf, buf):
            pltpu.async_copy(feature_hbm.at[ibuf.at[0]], buf, gsem).wait()

        def scatter(ibuf, buf):
            pltpu.async_copy(buf, acc.at[ibuf.at[1]], ssem, add=True)

        def wait_scatter(ibuf, buf):
            pltpu.make_async_copy(buf, acc.at[ibuf.at[1]], ssem).wait()

        fetch_idx(0, ibuf0)

        def body(i, carry):
            j = 2 * i
            wait_idx(ibuf0)
            gather(ibuf0, buf0)
            scatter(ibuf0, buf0)

            @pl.when(i > 0)
            def _():
                wait_scatter(ibuf1, buf1)

            fetch_idx(j + 1, ibuf1)
            wait_idx(ibuf1)
            gather(ibuf1, buf1)
            scatter(ibuf1, buf1)
            wait_scatter(ibuf0, buf0)
            fetch_idx(jnp.minimum(j + 2, n - 1), ibuf0)
            return carry

        lax.fori_loop(0, (n - 1) // 2, body, 0)
        wait_idx(ibuf0)
        gather(ibuf0, buf0)
        scatter(ibuf0, buf0)
        wait_scatter(ibuf1, buf1)
        wait_scatter(ibuf0, buf0)
        plsc.subcore_barrier()
        pltpu.sync_copy(acc.at[pl.ds(s * ZERO_ROWS, ZERO_ROWS)],
                        out_hbm.at[c, pl.ds(s * ZERO_ROWS, ZERO_ROWS)])

    return k(feature, ei, pads, zeros)


def _tc_linear(partials, W, b):
    blk = 1000
    grid = N_NODES // blk

    def body(p_ref, w_ref, b_ref, o_ref):
        x = p_ref[0] + p_ref[1]
        y = lax.dot_general(x, w_ref[...], (((1,), (1,)), ((), ())),
                            preferred_element_type=jnp.float32,
                            precision=lax.Precision.HIGHEST)
        o_ref[...] = y + b_ref[...]

    return pl.pallas_call(
        body,
        grid=(grid,),
        in_specs=[
            pl.BlockSpec((NUM_CORES, blk, D), lambda i: (0, i, 0)),
            pl.BlockSpec((D, D), lambda i: (0, 0)),
            pl.BlockSpec((1, D), lambda i: (0, 0)),
        ],
        out_specs=pl.BlockSpec((blk, D), lambda i: (i, 0)),
        out_shape=jax.ShapeDtypeStruct((N_NODES, D), jnp.float32),
    )(partials, W, b.reshape(1, D))


def kernel(feature, edge_index, W, b):
    ei = edge_index.astype(jnp.int32)
    pad = E_PAD - N_EDGES
    pad_src = jnp.arange(pad, dtype=jnp.int32) % N_NODES
    pad_dst = (N_NODES
               + jnp.arange(pad, dtype=jnp.int32) % (ACC_ROWS - N_NODES))
    pads = jnp.stack([pad_src, pad_dst])
    zeros = jnp.zeros((ZERO_ROWS, D), jnp.float32)
    partials = _sc_segment_sum(feature, ei, pads, zeros)
    return _tc_linear(partials, W, b)


# speedup vs baseline: 2.2503x; 1.3304x over previous
import functools

import jax
import jax.numpy as jnp
from jax import lax
from jax.experimental import pallas as pl
from jax.experimental.pallas import tpu as pltpu
from jax.experimental.pallas import tpu_sc as plsc

N_NODES = 10000
N_EDGES = 320000
D = 128

NUM_CORES = 2
NUM_SUBCORES = 16
NW = NUM_CORES * NUM_SUBCORES
CHUNK = 128
ROWS_C0 = 79
ROWS_C1 = 79
TOTAL_CHUNKS = NUM_SUBCORES * (ROWS_C0 + ROWS_C1)
E_PAD = TOTAL_CHUNKS * CHUNK
ACC_ROWS = 10240
ZERO_ROWS = ACC_ROWS // NUM_SUBCORES


def _sc_segment_sum(feature, ei, pads, zeros):
    mesh = plsc.VectorSubcoreMesh(core_axis_name="c", subcore_axis_name="s")

    @functools.partial(
        pl.kernel,
        out_type=jax.ShapeDtypeStruct((NUM_CORES, ACC_ROWS, D), jnp.float32),
        mesh=mesh,
        scratch_types=[
            pltpu.VMEM((2, CHUNK), jnp.int32),
            pltpu.VMEM((2, CHUNK), jnp.int32),
            pltpu.VMEM((CHUNK, D), jnp.float32),
            pltpu.VMEM((CHUNK, D), jnp.float32),
            pltpu.VMEM_SHARED((ACC_ROWS, D), jnp.float32),
            pltpu.SemaphoreType.DMA,
            pltpu.SemaphoreType.DMA,
            pltpu.SemaphoreType.DMA,
        ],
    )
    def k(feature_hbm, ei_hbm, pads_hbm, zeros_hbm, out_hbm,
          ibuf0, ibuf1, buf0, buf1, acc, isem, gsem, ssem):
        c = lax.axis_index("c")
        s = lax.axis_index("s")
        n = jnp.where(c == 0, ROWS_C0, ROWS_C1)
        off = jnp.where(c == 0, s * ROWS_C0,
                        NUM_SUBCORES * ROWS_C0 + s * ROWS_C1)
        pltpu.sync_copy(zeros_hbm, acc.at[pl.ds(s * ZERO_ROWS, ZERO_ROWS)])
        plsc.subcore_barrier()

        def fetch_idx(j, ibuf):
            e = (off + j) * CHUNK

            @pl.when(e < N_EDGES)
            def _():
                pltpu.async_copy(ei_hbm.at[0, pl.ds(e, CHUNK)],
                                 ibuf.at[0], isem)
                pltpu.async_copy(ei_hbm.at[1, pl.ds(e, CHUNK)],
                                 ibuf.at[1], isem)

            @pl.when(e >= N_EDGES)
            def _():
                ep = e - N_EDGES
                pltpu.async_copy(pads_hbm.at[0, pl.ds(ep, CHUNK)],
                                 ibuf.at[0], isem)
                pltpu.async_copy(pads_hbm.at[1, pl.ds(ep, CHUNK)],
                                 ibuf.at[1], isem)

        def wait_idx(ibuf):
            pltpu.make_async_copy(ei_hbm.at[0, pl.ds(0, CHUNK)],
                                  ibuf.at[0], isem).wait()
            pltpu.make_async_copy(ei_hbm.at[1, pl.ds(0, CHUNK)],
                                  ibuf.at[1], isem).wait()

        def gather(ibuf, buf):
            pltpu.async_copy(feature_hbm.at[ibuf.at[0]], buf, gsem).wait()

        def scatter(ibuf, buf):
            pltpu.async_copy(buf, acc.at[ibuf.at[1]], ssem, add=True)

        def wait_scatter(ibuf, buf):
            pltpu.make_async_copy(buf, acc.at[ibuf.at[1]], ssem).wait()

        fetch_idx(0, ibuf0)

        def body(i, carry):
            j = 2 * i
            wait_idx(ibuf0)
            gather(ibuf0, buf0)
            scatter(ibuf0, buf0)

            @pl.when(i > 0)
            def _():
                wait_scatter(ibuf1, buf1)

            fetch_idx(j + 1, ibuf1)
            wait_idx(ibuf1)
            gather(ibuf1, buf1)
            scatter(ibuf1, buf1)
            wait_scatter(ibuf0, buf0)
            fetch_idx(jnp.minimum(j + 2, n - 1), ibuf0)
            return carry

        lax.fori_loop(0, (n - 1) // 2, body, 0)
        wait_idx(ibuf0)
        gather(ibuf0, buf0)
        scatter(ibuf0, buf0)
        wait_scatter(ibuf1, buf1)
        wait_scatter(ibuf0, buf0)
        plsc.subcore_barrier()
        pltpu.sync_copy(acc.at[pl.ds(s * ZERO_ROWS, ZERO_ROWS)],
                        out_hbm.at[c, pl.ds(s * ZERO_ROWS, ZERO_ROWS)])

    return k(feature, ei, pads, zeros)


def _tc_linear(partials, W, b):
    blk = 1000
    grid = N_NODES // blk

    def body(p_ref, w_ref, b_ref, o_ref):
        x = p_ref[0] + p_ref[1]
        y = lax.dot_general(x, w_ref[...], (((1,), (1,)), ((), ())),
                            preferred_element_type=jnp.float32,
                            precision=lax.Precision.HIGHEST)
        o_ref[...] = y + b_ref[...]

    return pl.pallas_call(
        body,
        grid=(grid,),
        in_specs=[
            pl.BlockSpec((NUM_CORES, blk, D), lambda i: (0, i, 0)),
            pl.BlockSpec((D, D), lambda i: (0, 0)),
            pl.BlockSpec((1, D), lambda i: (0, 0)),
        ],
        out_specs=pl.BlockSpec((blk, D), lambda i: (i, 0)),
        out_shape=jax.ShapeDtypeStruct((N_NODES, D), jnp.float32),
    )(partials, W, b.reshape(1, D))


def kernel(feature, edge_index, W, b):
    ei = edge_index.astype(jnp.int32)
    pad = E_PAD - N_EDGES
    pad_src = jnp.arange(pad, dtype=jnp.int32) % N_NODES
    pad_dst = (N_NODES
               + jnp.arange(pad, dtype=jnp.int32) % (ACC_ROWS - N_NODES))
    pads = jnp.stack([pad_src, pad_dst])
    zeros = jnp.zeros((ZERO_ROWS, D), jnp.float32)
    partials = _sc_segment_sum(feature, ei, pads, zeros)
    return _tc_linear(partials, W, b)
